# Initial kernel scaffold; baseline (speedup 1.0000x reference)
import jax
import jax.numpy as jnp
from jax.experimental import pallas as pl


def kernel(x, spiral_x, W, b):
    raise NotImplementedError("write your pallas kernel here")


# speedup vs baseline: 4.3264x; 4.3264x over previous
import functools

import jax
import jax.numpy as jnp
from jax import lax
from jax.experimental import pallas as pl
from jax.experimental.pallas import tpu as pltpu
from jax.experimental.pallas import tpu_sc as plsc

N = 10000
F = 128
S = 16
OUT = 64

NC = 2
NS = 16
NW = NC * NS
VPW = 320
NPAD = NW * VPW
GV = 8
GROUP_ROWS = GV * S
NG = VPW // GV

MM_BLOCK = 2000


def _mm_body(x_ref, w_ref, o_ref):
    o_ref[...] = jnp.dot(x_ref[...], w_ref[...],
                         preferred_element_type=jnp.float32)


def _project(x2d, wt):
    return pl.pallas_call(
        _mm_body,
        grid=(N // MM_BLOCK,),
        in_specs=[
            pl.BlockSpec((MM_BLOCK, F), lambda i: (i, 0)),
            pl.BlockSpec((F, S * OUT), lambda i: (0, 0)),
        ],
        out_specs=pl.BlockSpec((MM_BLOCK, S * OUT), lambda i: (i, 0)),
        out_shape=jax.ShapeDtypeStruct((N, S * OUT), jnp.float32),
    )(x2d, wt)


def _sc_body(table_hbm, flat_hbm, b_hbm, out_hbm,
             idx_v, rows_v, out_v, bias_v, sem):
    wid = lax.axis_index("s") * NC + lax.axis_index("c")
    base_v = wid * VPW

    pltpu.sync_copy(b_hbm, bias_v)
    pltpu.sync_copy(flat_hbm.at[pl.ds(base_v * S, VPW * S)], idx_v)

    def group_body(g, carry):
        pltpu.async_copy(
            table_hbm.at[idx_v.at[pl.ds(g * GROUP_ROWS, GROUP_ROWS)]],
            rows_v, sem).wait()
        for j in range(GV):
            r0 = j * S
            for c in range(OUT // 16):
                acc = rows_v[r0, pl.ds(c * 16, 16)]
                for s in range(1, S):
                    acc = acc + rows_v[r0 + s, pl.ds(c * 16, 16)]
                acc = acc + bias_v[pl.ds(c * 16, 16)]
                acc = jnp.where(acc > 0.0, acc, jnp.exp(acc) - 1.0)
                out_v[g * GV + j, pl.ds(c * 16, 16)] = acc
        return carry

    lax.fori_loop(0, NG, group_body, 0)
    pltpu.sync_copy(out_v, out_hbm.at[pl.ds(base_v, VPW)])


@functools.cache
def _sc_gather():
    return functools.partial(
        pl.kernel,
        mesh=plsc.VectorSubcoreMesh(core_axis_name="c", subcore_axis_name="s"),
        compiler_params=pltpu.CompilerParams(use_tc_tiling_on_sc=False),
        out_type=jax.ShapeDtypeStruct((NPAD, OUT), jnp.float32),
        scratch_types=[
            pltpu.VMEM((VPW * S,), jnp.int32),
            pltpu.VMEM((GROUP_ROWS, OUT), jnp.float32),
            pltpu.VMEM((VPW, OUT), jnp.float32),
            pltpu.VMEM((OUT,), jnp.float32),
            pltpu.SemaphoreType.DMA,
        ],
    )(_sc_body)


def kernel(x, spiral_x, W, b):
    x2d = x.reshape(N, F)
    wt = W.reshape(OUT, S, F).transpose(2, 1, 0).reshape(F, S * OUT)
    table = _project(x2d, wt)

    sidx = spiral_x[0] * S + jnp.arange(S, dtype=jnp.int32)[None, :]
    flat = jnp.zeros((NPAD, S), jnp.int32).at[: N - 1].set(sidx).reshape(-1)

    out = _sc_gather()(table.reshape(N * S, OUT), flat, b)
    out = out[:N].at[N - 1].set(0.0)
    return out.reshape(1, N, OUT)
